# baseline (speedup 1.0000x reference)
import jax
import jax.numpy as jnp
from jax import lax
from jax.experimental import pallas as pl

_N = 1024
_C = 128
_NCHUNK = _N // _C
_ROWS_PER_BLOCK = 256


def _bitonic_body(x_ref, o_ref):
    chunks = [x_ref[:, v * _C:(v + 1) * _C] for v in range(_NCHUNK)]
    lanes = lax.broadcasted_iota(jnp.int32, (1, _C), 1)

    k = 2
    while k <= _N:
        j = k // 2
        while j >= 1:
            if j < _NCHUNK:
                if k < _NCHUNK:
                    asc_mask = None
                elif k < _N:
                    asc_mask = (lanes & (k >> 3)) == 0
                else:
                    asc_mask = None
                for v in range(_NCHUNK):
                    if v & j:
                        continue
                    w = v | j
                    mn = jnp.minimum(chunks[v], chunks[w])
                    mx = jnp.maximum(chunks[v], chunks[w])
                    if asc_mask is None:
                        asc = True if k == _N else (v & k) == 0
                        if asc:
                            chunks[v], chunks[w] = mn, mx
                        else:
                            chunks[v], chunks[w] = mx, mn
                    else:
                        chunks[v] = jnp.where(asc_mask, mn, mx)
                        chunks[w] = jnp.where(asc_mask, mx, mn)
            else:
                d = j // _NCHUNK
                low = (lanes & d) == 0
                if k == _N:
                    tm = low
                else:
                    tm = ((lanes & (k >> 3)) == 0) == low
                for v in range(_NCHUNK):
                    c = chunks[v]
                    p = jnp.where(low, jnp.roll(c, -d, axis=1),
                                  jnp.roll(c, d, axis=1))
                    mn = jnp.minimum(c, p)
                    mx = jnp.maximum(c, p)
                    chunks[v] = jnp.where(tm, mn, mx)
            j //= 2
        k *= 2

    for v in range(_NCHUNK):
        o_ref[:, v * _C:(v + 1) * _C] = chunks[v]


def kernel(x):
    b, t, n = x.shape
    rows = b * t
    x2 = x.reshape(rows, n)
    grid = rows // _ROWS_PER_BLOCK
    out = pl.pallas_call(
        _bitonic_body,
        out_shape=jax.ShapeDtypeStruct((rows, n), x.dtype),
        grid=(grid,),
        in_specs=[pl.BlockSpec((_ROWS_PER_BLOCK, n), lambda g: (g, 0))],
        out_specs=pl.BlockSpec((_ROWS_PER_BLOCK, n), lambda g: (g, 0)),
    )(x2)
    out = out.reshape(rows, _NCHUNK, _C).transpose(0, 2, 1)
    return out.reshape(b, t, n)


# speedup vs baseline: 1.2333x; 1.2333x over previous
import jax
import jax.numpy as jnp
from jax import lax
from jax.experimental import pallas as pl

_N = 1024
_C = 128
_NCHUNK = _N // _C
_ROWS_PER_BLOCK = 256


def _lane_bit(m):
    return 1 << m if m <= 6 else 1 << (m - 7)


def _bitonic_body(x_ref, o_ref):
    chunks = [x_ref[:, v * _C:(v + 1) * _C] for v in range(_NCHUNK)]
    lanes = lax.broadcasted_iota(jnp.int32, (1, _C), 1)

    for mk in range(1, 11):
        k = 1 << mk
        for mj in range(mk - 1, -1, -1):
            if mj < 3:
                jc = 1 << mj
                if mk < 3:
                    asc_mask = None
                elif mk < 10:
                    asc_mask = (lanes & _lane_bit(mk)) == 0
                else:
                    asc_mask = None
                for v in range(_NCHUNK):
                    if v & jc:
                        continue
                    w = v | jc
                    mn = jnp.minimum(chunks[v], chunks[w])
                    mx = jnp.maximum(chunks[v], chunks[w])
                    if asc_mask is None:
                        asc = True if mk == 10 else (v & k) == 0
                        if asc:
                            chunks[v], chunks[w] = mn, mx
                        else:
                            chunks[v], chunks[w] = mx, mn
                    else:
                        chunks[v] = jnp.where(asc_mask, mn, mx)
                        chunks[w] = jnp.where(asc_mask, mx, mn)
            else:
                d = _lane_bit(mj)
                low = (lanes & d) == 0
                if mk == 10:
                    tm = low
                else:
                    tm = ((lanes & _lane_bit(mk)) == 0) == low
                for v in range(_NCHUNK):
                    c = chunks[v]
                    p = jnp.where(low, jnp.roll(c, -d, axis=1),
                                  jnp.roll(c, d, axis=1))
                    mn = jnp.minimum(c, p)
                    mx = jnp.maximum(c, p)
                    chunks[v] = jnp.where(tm, mn, mx)

    for p in range(3):
        d = 1 << p
        lbit = (lanes & d) != 0
        for v in range(_NCHUNK):
            if v & d:
                continue
            w = v | d
            lo, hi = chunks[v], chunks[w]
            chunks[v] = jnp.where(lbit, jnp.roll(hi, d, axis=1), lo)
            chunks[w] = jnp.where(lbit, hi, jnp.roll(lo, -d, axis=1))

    for v in range(_NCHUNK):
        o_ref[:, v * _C:(v + 1) * _C] = chunks[v]


def kernel(x):
    b, t, n = x.shape
    rows = b * t
    x2 = x.reshape(rows, n)
    grid = rows // _ROWS_PER_BLOCK
    out = pl.pallas_call(
        _bitonic_body,
        out_shape=jax.ShapeDtypeStruct((rows, n), x.dtype),
        grid=(grid,),
        in_specs=[pl.BlockSpec((_ROWS_PER_BLOCK, n), lambda g: (g, 0))],
        out_specs=pl.BlockSpec((_ROWS_PER_BLOCK, n), lambda g: (g, 0)),
    )(x2)
    return out.reshape(b, t, n)
